# baseline (speedup 1.0000x reference)
import jax
import jax.numpy as jnp
from jax import lax
from jax.experimental import pallas as pl
from jax.experimental.pallas import tpu as pltpu
from jax.experimental.pallas import tpu_sc as plsc

B = 64
F = 16384
O = 16384
NNZ = 268435
G = 8
Q = F // G
GRID_MIN = -2.0
GRID_MAX = 2.0
GRID_STEP = (GRID_MAX - GRID_MIN) / (G - 1)
INV_DENOM = 1.0 / GRID_STEP
TW = G * B + B

NC = 2
NS = 16
NT = NC * NS
E_T = 8448
EPAD = NT * E_T
C = 16
NCHUNK = E_T // C
DEPTH = 4
AWC = C * G + C

QB = 256


def _table_body(xg_ref, xkq_ref, tab_ref):
    k = pl.program_id(0)
    gr = GRID_MIN + k.astype(jnp.float32) * GRID_STEP
    t = (xg_ref[...] - gr) * INV_DENOM
    tab_ref[:, : G * B] = jnp.exp(-(t * t))
    sx = xkq_ref[...]
    tab_ref[:, G * B :] = sx * jax.nn.sigmoid(sx)


def _sc_body(tab_h, cols_h, rows_h, aux_h, part_h,
             acc_s, cols_v, xbs0, xbs1, xbs2, xbs3, axs0, axs1, axs2, axs3,
             rws0, rws1, rws2, rws3, res0, res1, sg0, sg1, sg2, sg3, ssc):
    cid = lax.axis_index("c")
    sid = lax.axis_index("s")
    tid = cid * NS + sid
    ebase = tid * E_T
    cbase = tid * NCHUNK
    xbs = (xbs0, xbs1, xbs2, xbs3)
    axs = (axs0, axs1, axs2, axs3)
    rws = (rws0, rws1, rws2, rws3)
    sgs = (sg0, sg1, sg2, sg3)
    ress = (res0, res1)

    pltpu.sync_copy(cols_h.at[pl.ds(ebase, E_T)], cols_v)

    zv = jnp.zeros((16,), jnp.float32)
    for i in range(C):
        for v in range(B // 16):
            res0[i, pl.ds(v * 16, 16)] = zv
    rows_per_tile = O // NS
    rbase = sid * rows_per_tile

    def zbody(t, _):
        pltpu.sync_copy(res0, acc_s.at[pl.ds(rbase + t * C, C)])
        return 0

    lax.fori_loop(0, rows_per_tile // C, zbody, 0)
    plsc.subcore_barrier()

    def fire(ci, p):
        off = ci * C
        cvec = cols_v[pl.ds(off, C)]
        jvec = jnp.bitwise_and(cvec, 7) * Q + lax.shift_right_logical(cvec, 3)
        pltpu.async_copy(tab_h.at[jvec], xbs[p], sgs[p])
        pltpu.async_copy(aux_h.at[pl.ds((cbase + ci) * AWC, AWC)], axs[p], sgs[p])
        pltpu.async_copy(rows_h.at[pl.ds(off + ebase, C)], rws[p], sgs[p])

    def compute(ci, p, rp, first):
        xb = xbs[p]
        ax = axs[p]
        res = ress[rp]
        pltpu.make_async_copy(tab_h.at[pl.ds(0, C)], xb, sgs[p]).wait()
        pltpu.make_async_copy(aux_h.at[pl.ds(0, AWC)], ax, sgs[p]).wait()
        pltpu.make_async_copy(rows_h.at[pl.ds(0, C)], rws[p], sgs[p]).wait()

        bwvec = ax[pl.ds(C * G, 16)]
        for i in range(C):
            if i % 2 == 0:
                swv = ax[pl.ds(i * G, 16)]
            wb = (i % 2) * G
            bw = bwvec[i]
            for v in range(B // 16):
                accv = bw * xb[i, pl.ds(G * B + v * 16, 16)]
                for g in range(G):
                    accv = accv + swv[wb + g] * xb[i, pl.ds(g * B + v * 16, 16)]
                res[i, pl.ds(v * 16, 16)] = accv

        rvec = rws[p][...]
        if first is None:
            pltpu.make_async_copy(part_h.at[0, pl.ds(0, C)], res, ssc).wait()
        else:
            @pl.when(jnp.logical_not(first))
            def _():
                pltpu.make_async_copy(part_h.at[0, pl.ds(0, C)], res, ssc).wait()
        pltpu.async_copy(res, acc_s.at[rvec], ssc, add=True)

    for ci in range(DEPTH - 1):
        fire(ci, ci)

    def quad_body(pi, _):
        ci0 = pi * DEPTH
        for s in range(DEPTH):
            ci = ci0 + s
            fire(jnp.minimum(ci + DEPTH - 1, NCHUNK - 1), (s + DEPTH - 1) % DEPTH)
            compute(ci, s, s % 2, (pi == 0) if s == 0 else None)
        return 0

    lax.fori_loop(0, NCHUNK // DEPTH, quad_body, 0)

    pltpu.make_async_copy(part_h.at[0, pl.ds(0, C)], res0, ssc).wait()
    for p in range(DEPTH - 1):
        pltpu.make_async_copy(tab_h.at[pl.ds(0, C)], xbs[p], sgs[p]).wait()
        pltpu.make_async_copy(aux_h.at[pl.ds(0, AWC)], axs[p], sgs[p]).wait()
        pltpu.make_async_copy(rows_h.at[pl.ds(0, C)], rws[p], sgs[p]).wait()

    plsc.subcore_barrier()
    pltpu.sync_copy(acc_s.at[pl.ds(rbase, rows_per_tile)],
                    part_h.at[cid, pl.ds(rbase, rows_per_tile)])


def _combine_body(part_ref, out_ref):
    s = part_ref[0] + part_ref[1]
    out_ref[...] = s.T


def kernel(x, conn_rows, conn_cols, spline_w, base_w):
    xg = x.reshape(B, G, Q).transpose(2, 1, 0).reshape(Q, G * B)
    xkq = x.T.reshape(Q, G, B).transpose(1, 0, 2).reshape(F, B)
    pad = EPAD - NNZ
    rows_p = jnp.concatenate([conn_rows, jnp.zeros((pad,), jnp.int32)])
    cols_p = jnp.concatenate([conn_cols, jnp.zeros((pad,), jnp.int32)])
    sw_p = jnp.concatenate([spline_w, jnp.zeros((pad, G), jnp.float32)])
    bw_p = jnp.concatenate([base_w, jnp.zeros((pad,), jnp.float32)])
    nchunk_tot = EPAD // C
    aux = jnp.concatenate(
        [sw_p.reshape(nchunk_tot, C * G), bw_p.reshape(nchunk_tot, C)],
        axis=1).reshape(nchunk_tot * AWC)

    tab = pl.pallas_call(
        _table_body,
        grid=(G, Q // QB),
        in_specs=[
            pl.BlockSpec((QB, G * B), lambda k, i: (i, 0)),
            pl.BlockSpec((QB, B), lambda k, i: (k * (Q // QB) + i, 0)),
        ],
        out_specs=pl.BlockSpec((QB, TW), lambda k, i: (k * (Q // QB) + i, 0)),
        out_shape=jax.ShapeDtypeStruct((F, TW), jnp.float32),
    )(xg, xkq)

    mesh = plsc.VectorSubcoreMesh(core_axis_name="c", subcore_axis_name="s")
    sc_call = pl.kernel(
        _sc_body,
        out_type=jax.ShapeDtypeStruct((NC, O, B), jnp.float32),
        mesh=mesh,
        scratch_types=(
            [pltpu.VMEM_SHARED((O, B), jnp.float32),
             pltpu.VMEM((E_T,), jnp.int32)]
            + [pltpu.VMEM((C, TW), jnp.float32) for _ in range(DEPTH)]
            + [pltpu.VMEM((AWC,), jnp.float32) for _ in range(DEPTH)]
            + [pltpu.VMEM((C,), jnp.int32) for _ in range(DEPTH)]
            + [pltpu.VMEM((C, B), jnp.float32) for _ in range(2)]
            + [pltpu.SemaphoreType.DMA for _ in range(DEPTH + 1)]
        ),
        compiler_params=pltpu.CompilerParams(use_tc_tiling_on_sc=False),
    )
    part = sc_call(tab, cols_p, rows_p, aux)

    BLK = 512
    out = pl.pallas_call(
        _combine_body,
        grid=(O // BLK,),
        in_specs=[pl.BlockSpec((NC, BLK, B), lambda i: (0, i, 0))],
        out_specs=pl.BlockSpec((B, BLK), lambda i: (0, i)),
        out_shape=jax.ShapeDtypeStruct((B, O), jnp.float32),
    )(part)
    return out


# speedup vs baseline: 1.2817x; 1.2817x over previous
import jax
import jax.numpy as jnp
from jax import lax
from jax.experimental import pallas as pl
from jax.experimental.pallas import tpu as pltpu
from jax.experimental.pallas import tpu_sc as plsc

B = 64
F = 16384
O = 16384
NNZ = 268435
G = 8
Q = F // G
GRID_MIN = -2.0
GRID_MAX = 2.0
GRID_STEP = (GRID_MAX - GRID_MIN) / (G - 1)
INV_DENOM = 1.0 / GRID_STEP
TW = G * B + B

NC = 2
NS = 16
NT = NC * NS
E_T = 8448
EPAD = NT * E_T
C = 16
NCHUNK = E_T // C
DEPTH = 2
AWC = C * G + C

QB = 256

ILV = plsc.PackFormat.INTERLEAVED


def _table_body(xg_ref, xkq_ref, tab_ref):
    k = pl.program_id(0)
    gr = GRID_MIN + k.astype(jnp.float32) * GRID_STEP
    t = (xg_ref[...] - gr) * INV_DENOM
    tab_ref[:, : G * B] = jnp.exp(-(t * t)).astype(jnp.bfloat16)
    sx = xkq_ref[...]
    tab_ref[:, G * B :] = (sx * jax.nn.sigmoid(sx)).astype(jnp.bfloat16)


def _sc_body(tab_h, cols_h, rows_h, aux_h, part_h,
             acc_s, cols_v, xbs0, xbs1, axs0, axs1,
             rws0, rws1, res0, res1, sg0, sg1, ssc):
    cid = lax.axis_index("c")
    sid = lax.axis_index("s")
    tid = cid * NS + sid
    ebase = tid * E_T
    cbase = tid * NCHUNK
    xbs = (xbs0, xbs1)
    axs = (axs0, axs1)
    rws = (rws0, rws1)
    sgs = (sg0, sg1)
    ress = (res0, res1)

    pltpu.sync_copy(cols_h.at[pl.ds(ebase, E_T)], cols_v)

    zv = jnp.zeros((16,), jnp.float32)
    for i in range(C):
        for v in range(B // 16):
            res0[i, pl.ds(v * 16, 16)] = zv
    rows_per_tile = O // NS
    rbase = sid * rows_per_tile

    def zbody(t, _):
        pltpu.sync_copy(res0, acc_s.at[pl.ds(rbase + t * C, C)])
        return 0

    lax.fori_loop(0, rows_per_tile // C, zbody, 0)
    plsc.subcore_barrier()

    def fire(ci, p):
        off = ci * C
        cvec = cols_v[pl.ds(off, C)]
        jvec = jnp.bitwise_and(cvec, 7) * Q + lax.shift_right_logical(cvec, 3)
        pltpu.async_copy(tab_h.at[jvec], xbs[p], sgs[p])
        pltpu.async_copy(aux_h.at[pl.ds((cbase + ci) * AWC, AWC)], axs[p], sgs[p])
        pltpu.async_copy(rows_h.at[pl.ds(off + ebase, C)], rws[p], sgs[p])

    def compute(ci, p, rp, first):
        xb = xbs[p]
        ax = axs[p]
        res = ress[rp]
        pltpu.make_async_copy(tab_h.at[pl.ds(0, C)], xb, sgs[p]).wait()
        pltpu.make_async_copy(aux_h.at[pl.ds(0, AWC)], ax, sgs[p]).wait()
        pltpu.make_async_copy(rows_h.at[pl.ds(0, C)], rws[p], sgs[p]).wait()

        bwvec = ax[pl.ds(C * G, 16)]
        for i in range(C):
            if i % 2 == 0:
                swv = ax[pl.ds(i * G, 16)]
            wb = (i % 2) * G
            bw = bwvec[i]
            for h in range(2):
                sl, sh = plsc.unpack(xb[i, pl.ds(G * B + h * 32, 32)], format=ILV)
                acc_l = bw * sl
                acc_h = bw * sh
                for g in range(G):
                    tl, th = plsc.unpack(xb[i, pl.ds(g * B + h * 32, 32)], format=ILV)
                    w = swv[wb + g]
                    acc_l = acc_l + w * tl
                    acc_h = acc_h + w * th
                res[i, pl.ds(h * 32, 16)] = acc_l
                res[i, pl.ds(h * 32 + 16, 16)] = acc_h

        rvec = rws[p][...]
        if first is None:
            pltpu.make_async_copy(part_h.at[0, pl.ds(0, C)], res, ssc).wait()
        else:
            @pl.when(jnp.logical_not(first))
            def _():
                pltpu.make_async_copy(part_h.at[0, pl.ds(0, C)], res, ssc).wait()
        pltpu.async_copy(res, acc_s.at[rvec], ssc, add=True)

    for ci in range(DEPTH - 1):
        fire(ci, ci)

    def pair_body(pi, _):
        ci0 = pi * DEPTH
        for s in range(DEPTH):
            ci = ci0 + s
            fire(jnp.minimum(ci + DEPTH - 1, NCHUNK - 1), (s + DEPTH - 1) % DEPTH)
            compute(ci, s, s % 2, (pi == 0) if s == 0 else None)
        return 0

    lax.fori_loop(0, NCHUNK // DEPTH, pair_body, 0)

    pltpu.make_async_copy(part_h.at[0, pl.ds(0, C)], res0, ssc).wait()
    for p in range(DEPTH - 1):
        pltpu.make_async_copy(tab_h.at[pl.ds(0, C)], xbs[p], sgs[p]).wait()
        pltpu.make_async_copy(aux_h.at[pl.ds(0, AWC)], axs[p], sgs[p]).wait()
        pltpu.make_async_copy(rows_h.at[pl.ds(0, C)], rws[p], sgs[p]).wait()

    plsc.subcore_barrier()
    pltpu.sync_copy(acc_s.at[pl.ds(rbase, rows_per_tile)],
                    part_h.at[cid, pl.ds(rbase, rows_per_tile)])


def _combine_body(part_ref, out_ref):
    s = part_ref[0] + part_ref[1]
    out_ref[...] = s.T


def kernel(x, conn_rows, conn_cols, spline_w, base_w):
    xg3 = x.reshape(B, G, Q).transpose(2, 1, 0)
    xgp = xg3.reshape(Q, G, 2, 2, 16).transpose(0, 1, 2, 4, 3).reshape(Q, G * B)
    xkq = x.T.reshape(Q, G, B).transpose(1, 0, 2).reshape(F, B)
    xkqp = xkq.reshape(F, 2, 2, 16).transpose(0, 1, 3, 2).reshape(F, B)
    pad = EPAD - NNZ
    rows_p = jnp.concatenate([conn_rows, jnp.zeros((pad,), jnp.int32)])
    cols_p = jnp.concatenate([conn_cols, jnp.zeros((pad,), jnp.int32)])
    sw_p = jnp.concatenate([spline_w, jnp.zeros((pad, G), jnp.float32)])
    bw_p = jnp.concatenate([base_w, jnp.zeros((pad,), jnp.float32)])
    nchunk_tot = EPAD // C
    aux = jnp.concatenate(
        [sw_p.reshape(nchunk_tot, C * G), bw_p.reshape(nchunk_tot, C)],
        axis=1).reshape(nchunk_tot * AWC)

    tab = pl.pallas_call(
        _table_body,
        grid=(G, Q // QB),
        in_specs=[
            pl.BlockSpec((QB, G * B), lambda k, i: (i, 0)),
            pl.BlockSpec((QB, B), lambda k, i: (k * (Q // QB) + i, 0)),
        ],
        out_specs=pl.BlockSpec((QB, TW), lambda k, i: (k * (Q // QB) + i, 0)),
        out_shape=jax.ShapeDtypeStruct((F, TW), jnp.bfloat16),
    )(xgp, xkqp)

    mesh = plsc.VectorSubcoreMesh(core_axis_name="c", subcore_axis_name="s")
    sc_call = pl.kernel(
        _sc_body,
        out_type=jax.ShapeDtypeStruct((NC, O, B), jnp.float32),
        mesh=mesh,
        scratch_types=(
            [pltpu.VMEM_SHARED((O, B), jnp.float32),
             pltpu.VMEM((E_T,), jnp.int32)]
            + [pltpu.VMEM((C, TW), jnp.bfloat16) for _ in range(DEPTH)]
            + [pltpu.VMEM((AWC,), jnp.float32) for _ in range(DEPTH)]
            + [pltpu.VMEM((C,), jnp.int32) for _ in range(DEPTH)]
            + [pltpu.VMEM((C, B), jnp.float32) for _ in range(2)]
            + [pltpu.SemaphoreType.DMA for _ in range(DEPTH + 1)]
        ),
        compiler_params=pltpu.CompilerParams(use_tc_tiling_on_sc=False,
                                             needs_layout_passes=False),
    )
    part = sc_call(tab, cols_p, rows_p, aux)

    BLK = 512
    out = pl.pallas_call(
        _combine_body,
        grid=(O // BLK,),
        in_specs=[pl.BlockSpec((NC, BLK, B), lambda i: (0, i, 0))],
        out_specs=pl.BlockSpec((B, BLK), lambda i: (0, i)),
        out_shape=jax.ShapeDtypeStruct((B, O), jnp.float32),
    )(part)
    return out
